# baseline (speedup 1.0000x reference)
import jax
import jax.numpy as jnp
from jax.experimental import pallas as pl
from jax.experimental.pallas import tpu as pltpu

_B, _T, _D = 16384, 100, 64
_LANES = 128
_MROWS = (_B * _T) // _LANES


def _body(x_hbm, y_hbm, mask_ref):
    pltpu.sync_copy(x_hbm, y_hbm)
    mask_ref[...] = jnp.zeros_like(mask_ref)


def kernel(inputs):
    y, mask2d = pl.pallas_call(
        _body,
        out_shape=(
            jax.ShapeDtypeStruct((_B, _T, _D), inputs.dtype),
            jax.ShapeDtypeStruct((_MROWS, _LANES), inputs.dtype),
        ),
        in_specs=[pl.BlockSpec(memory_space=pltpu.MemorySpace.HBM)],
        out_specs=(
            pl.BlockSpec(memory_space=pltpu.MemorySpace.HBM),
            pl.BlockSpec((_MROWS, _LANES), lambda: (0, 0)),
        ),
    )(inputs)
    return (y, mask2d.reshape(_B, _T, 1))


# speedup vs baseline: 15.4575x; 15.4575x over previous
import jax
import jax.numpy as jnp
from jax.experimental import pallas as pl
from jax.experimental.pallas import tpu as pltpu

_B, _T, _D = 16384, 100, 64
_LANES = 128
_MROWS = (_B * _T) // _LANES
_NBUF = 8
_AHEAD = _NBUF // 2
_CB = 128
_NC = _B // _CB


def _body(x_hbm, y_hbm, mask_ref, bufs, sin, sout):
    def in_copy(c, s):
        return pltpu.make_async_copy(
            x_hbm.at[pl.ds(c * _CB, _CB)], bufs.at[s], sin.at[s])

    def out_copy(c, s):
        return pltpu.make_async_copy(
            bufs.at[s], y_hbm.at[pl.ds(c * _CB, _CB)], sout.at[s])

    for c in range(_AHEAD):
        in_copy(c, c % _NBUF).start()
    mask_ref[...] = jnp.zeros_like(mask_ref)
    for c in range(_NC):
        s = c % _NBUF
        in_copy(c, s).wait()
        out_copy(c, s).start()
        nxt = c + _AHEAD
        if nxt < _NC:
            sn = nxt % _NBUF
            if nxt >= _NBUF:
                out_copy(nxt - _NBUF, sn).wait()
            in_copy(nxt, sn).start()
    for c in range(_NC - _NBUF, _NC):
        if c >= 0:
            out_copy(c, c % _NBUF).wait()


def kernel(inputs):
    y, mask2d = pl.pallas_call(
        _body,
        out_shape=(
            jax.ShapeDtypeStruct((_B, _T, _D), inputs.dtype),
            jax.ShapeDtypeStruct((_MROWS, _LANES), inputs.dtype),
        ),
        in_specs=[pl.BlockSpec(memory_space=pltpu.MemorySpace.HBM)],
        out_specs=(
            pl.BlockSpec(memory_space=pltpu.MemorySpace.HBM),
            pl.BlockSpec((_MROWS, _LANES), lambda: (0, 0)),
        ),
        scratch_shapes=[
            pltpu.VMEM((_NBUF, _CB, _T, _D), jnp.float32),
            pltpu.SemaphoreType.DMA((_NBUF,)),
            pltpu.SemaphoreType.DMA((_NBUF,)),
        ],
    )(inputs)
    return (y, mask2d.reshape(_B, _T, 1))


# speedup vs baseline: 25.7536x; 1.6661x over previous
import jax
import jax.numpy as jnp
from jax.experimental import pallas as pl

_B, _T, _D = 16384, 100, 64
_F = _T * _D
_LANES = 128
_MROWS = (_B * _T) // _LANES
_GRID = 64
_BB = _B // _GRID
_MB = _MROWS // _GRID


def _body(x_ref, y_ref, mask_ref):
    y_ref[...] = x_ref[...]
    mask_ref[...] = jnp.zeros_like(mask_ref)


def kernel(inputs):
    x2 = inputs.reshape(_B, _F)
    y2, mask2d = pl.pallas_call(
        _body,
        out_shape=(
            jax.ShapeDtypeStruct((_B, _F), inputs.dtype),
            jax.ShapeDtypeStruct((_MROWS, _LANES), inputs.dtype),
        ),
        grid=(_GRID,),
        in_specs=[pl.BlockSpec((_BB, _F), lambda i: (i, 0))],
        out_specs=(
            pl.BlockSpec((_BB, _F), lambda i: (i, 0)),
            pl.BlockSpec((_MB, _LANES), lambda i: (i, 0)),
        ),
    )(x2)
    return (y2.reshape(_B, _T, _D), mask2d.reshape(_B, _T, 1))
